# baseline (speedup 1.0000x reference)
import functools

import jax
import jax.numpy as jnp
from jax import lax
from jax.experimental import pallas as pl
from jax.experimental.pallas import tpu as pltpu
from jax.experimental.pallas import tpu_sc as plsc

K = 8192
D = 256
BM = 1024


def _csq_body(cb_ref, csq_ref):
    cb = cb_ref[...]
    csq_ref[...] = jnp.sum(cb * cb, axis=1)[None, :]


GR = 8


def _argmin_tc_body(q_ref, cb_ref, csq_ref, iota_ref, idx_ref, s_ref):
    s_ref[...] = lax.dot_general(
        q_ref[...], cb_ref[...], (((1,), (1,)), ((), ())),
        preferred_element_type=jnp.float32)

    def group(g, carry):
        qg = q_ref[pl.ds(g * GR, GR), :]
        zsq_g = jnp.sum(qg * qg, axis=1, keepdims=True)
        d = (zsq_g + csq_ref[...]) - 2.0 * s_ref[pl.ds(g * GR, GR), :]
        m = jnp.min(d, axis=1, keepdims=True)
        idx_f = jnp.min(
            jnp.where(d == m, iota_ref[...], jnp.float32(K)),
            axis=1, keepdims=True)
        idx_ref[pl.ds(g * GR, GR), :] = idx_f.astype(jnp.int32)
        return carry

    lax.fori_loop(0, BM // GR, group, 0)


def _codebook_norms(codebook):
    return pl.pallas_call(
        _csq_body,
        out_shape=jax.ShapeDtypeStruct((1, K), jnp.float32),
    )(codebook)


def _argmin_indices(q, codebook, csq):
    m = q.shape[0]
    iota = jnp.arange(K, dtype=jnp.float32)[None, :]
    return pl.pallas_call(
        _argmin_tc_body,
        grid=(m // BM,),
        in_specs=[
            pl.BlockSpec((BM, D), lambda i: (i, 0)),
            pl.BlockSpec((K, D), lambda i: (0, 0)),
            pl.BlockSpec((1, K), lambda i: (0, 0)),
            pl.BlockSpec((1, K), lambda i: (0, 0)),
        ],
        out_specs=pl.BlockSpec((BM, 1), lambda i: (i, 0)),
        out_shape=jax.ShapeDtypeStruct((m, 1), jnp.int32),
        scratch_shapes=[pltpu.VMEM((BM, K), jnp.float32)],
        compiler_params=pltpu.CompilerParams(
            dimension_semantics=("parallel",)),
    )(q, codebook, csq, iota).reshape(m)


def _make_gather(b_total):
    info = plsc.get_sparse_core_info()
    nc, ns = info.num_cores, info.num_subcores
    nw = nc * ns
    b_per_w = b_total // nw
    chunk = 128
    n_chunks = b_per_w // chunk
    mesh = plsc.VectorSubcoreMesh(core_axis_name="c", subcore_axis_name="s")

    @functools.partial(
        pl.kernel, mesh=mesh,
        out_type=jax.ShapeDtypeStruct((b_total, D), jnp.float32),
        scratch_types=[
            pltpu.VMEM((b_per_w,), jnp.int32),
            pltpu.VMEM((chunk, D), jnp.float32),
            pltpu.VMEM((chunk, D), jnp.float32),
            pltpu.SemaphoreType.DMA,
            pltpu.SemaphoreType.DMA,
        ],
    )
    def gather_k(table_hbm, idx_hbm, out_hbm, idx_v, rows0, rows1, sem0, sem1):
        wid = lax.axis_index("s") * nc + lax.axis_index("c")
        base = wid * b_per_w
        pltpu.sync_copy(idx_hbm.at[pl.ds(base, b_per_w)], idx_v)
        bufs = (rows0, rows1)
        sems = (sem0, sem1)

        def start(i):
            return pltpu.async_copy(
                table_hbm.at[idx_v.at[pl.ds(i * chunk, chunk)]],
                bufs[i % 2], sems[i % 2])

        cps = {0: start(0)}
        for i in range(n_chunks):
            if i + 1 < n_chunks:
                cps[i + 1] = start(i + 1)
            cps[i].wait()
            pltpu.sync_copy(bufs[i % 2],
                            out_hbm.at[pl.ds(base + i * chunk, chunk)])

    return gather_k


NSPLIT = 1


def kernel(ze, codebook):
    b, t, f = ze.shape
    m = b * t * (f // D)
    q = ze.reshape(m, D)
    ms = m // NSPLIT
    csq = _codebook_norms(codebook)
    gather = _make_gather(ms)
    parts = []
    for s in range(NSPLIT):
        qs = lax.slice_in_dim(q, s * ms, (s + 1) * ms, axis=0)
        idx = _argmin_indices(qs, codebook, csq)
        parts.append(gather(codebook, idx))
    zq = parts[0] if len(parts) == 1 else jnp.concatenate(parts, axis=0)
    return zq.reshape(b, t, f)


# speedup vs baseline: 3.2219x; 3.2219x over previous
import functools

import jax
import jax.numpy as jnp
from jax import lax
from jax.experimental import pallas as pl
from jax.experimental.pallas import tpu as pltpu
from jax.experimental.pallas import tpu_sc as plsc

K = 8192
D = 256
BM = 1024


def _csq_body(cb_ref, csq_ref):
    cb = cb_ref[...]
    csq_ref[...] = jnp.sum(cb * cb, axis=1)[None, :]


def _argmin_tc_body(q_ref, cb_ref, csq_ref, iota_ref, idx_ref):
    q = q_ref[...]
    scores = lax.dot_general(
        q, cb_ref[...], (((1,), (1,)), ((), ())),
        preferred_element_type=jnp.float32)
    zsq = jnp.sum(q * q, axis=1, keepdims=True)
    d = (zsq + csq_ref[...]) - 2.0 * scores
    m = jnp.min(d, axis=1, keepdims=True)
    idx_f = jnp.min(jnp.where(d == m, iota_ref[...], jnp.float32(K)), axis=1)
    idx_ref[...] = idx_f.astype(jnp.int32)


def _codebook_norms(codebook):
    return pl.pallas_call(
        _csq_body,
        out_shape=jax.ShapeDtypeStruct((1, K), jnp.float32),
    )(codebook)


def _argmin_indices(q, codebook, csq):
    m = q.shape[0]
    iota = jnp.arange(K, dtype=jnp.float32)[None, :]
    return pl.pallas_call(
        _argmin_tc_body,
        grid=(m // BM,),
        in_specs=[
            pl.BlockSpec((BM, D), lambda i: (i, 0)),
            pl.BlockSpec((K, D), lambda i: (0, 0)),
            pl.BlockSpec((1, K), lambda i: (0, 0)),
            pl.BlockSpec((1, K), lambda i: (0, 0)),
        ],
        out_specs=pl.BlockSpec((BM,), lambda i: (i,)),
        out_shape=jax.ShapeDtypeStruct((m,), jnp.int32),
        compiler_params=pltpu.CompilerParams(
            dimension_semantics=("arbitrary",)),
    )(q, codebook, csq, iota)


def _make_gather(b_total):
    info = plsc.get_sparse_core_info()
    nc, ns = info.num_cores, info.num_subcores
    nw = nc * ns
    b_per_w = b_total // nw
    chunk = 128
    n_chunks = b_per_w // chunk
    mesh = plsc.VectorSubcoreMesh(core_axis_name="c", subcore_axis_name="s")

    @functools.partial(
        pl.kernel, mesh=mesh,
        out_type=jax.ShapeDtypeStruct((b_total, D), jnp.float32),
        scratch_types=[
            pltpu.VMEM((b_per_w,), jnp.int32),
            pltpu.VMEM((chunk, D), jnp.float32),
            pltpu.VMEM((chunk, D), jnp.float32),
            pltpu.SemaphoreType.DMA,
            pltpu.SemaphoreType.DMA,
        ],
    )
    def gather_k(table_hbm, idx_hbm, out_hbm, idx_v, rows0, rows1, sem0, sem1):
        wid = lax.axis_index("s") * nc + lax.axis_index("c")
        base = wid * b_per_w
        pltpu.sync_copy(idx_hbm.at[pl.ds(base, b_per_w)], idx_v)
        bufs = (rows0, rows1)
        sems = (sem0, sem1)

        def start(i):
            return pltpu.async_copy(
                table_hbm.at[idx_v.at[pl.ds(i * chunk, chunk)]],
                bufs[i % 2], sems[i % 2])

        cps = {0: start(0)}
        for i in range(n_chunks):
            if i + 1 < n_chunks:
                cps[i + 1] = start(i + 1)
            cps[i].wait()
            pltpu.sync_copy(bufs[i % 2],
                            out_hbm.at[pl.ds(base + i * chunk, chunk)])

    return gather_k


NSPLIT = 1


def kernel(ze, codebook):
    b, t, f = ze.shape
    m = b * t * (f // D)
    q = ze.reshape(m, D)
    ms = m // NSPLIT
    csq = _codebook_norms(codebook)
    gather = _make_gather(ms)
    parts = []
    for s in range(NSPLIT):
        qs = lax.slice_in_dim(q, s * ms, (s + 1) * ms, axis=0)
        idx = _argmin_indices(qs, codebook, csq)
        parts.append(gather(codebook, idx))
    zq = parts[0] if len(parts) == 1 else jnp.concatenate(parts, axis=0)
    return zq.reshape(b, t, f)


# speedup vs baseline: 3.2593x; 1.0116x over previous
import functools

import jax
import jax.numpy as jnp
from jax import lax
from jax.experimental import pallas as pl
from jax.experimental.pallas import tpu as pltpu
from jax.experimental.pallas import tpu_sc as plsc

K = 8192
D = 256
BM = 1024


def _csq_body(cb_ref, csq_ref):
    cb = cb_ref[...]
    csq_ref[...] = jnp.sum(cb * cb, axis=1)[None, :]


def _argmin_tc_body(q_ref, cb_ref, iota_ref, idx_ref, csq_ref):
    @pl.when(pl.program_id(0) == 0)
    def _():
        cb = cb_ref[...]
        csq_ref[...] = jnp.sum(cb * cb, axis=1)[None, :]

    q = q_ref[...]
    scores = lax.dot_general(
        q, cb_ref[...], (((1,), (1,)), ((), ())),
        preferred_element_type=jnp.float32)
    zsq = jnp.sum(q * q, axis=1, keepdims=True)
    d = (zsq + csq_ref[...]) - 2.0 * scores
    m = jnp.min(d, axis=1, keepdims=True)
    idx_f = jnp.min(jnp.where(d == m, iota_ref[...], jnp.float32(K)), axis=1)
    idx_ref[...] = idx_f.astype(jnp.int32)


def _codebook_norms(codebook):
    return pl.pallas_call(
        _csq_body,
        out_shape=jax.ShapeDtypeStruct((1, K), jnp.float32),
    )(codebook)


def _argmin_indices(q, codebook):
    m = q.shape[0]
    iota = jnp.arange(K, dtype=jnp.float32)[None, :]
    return pl.pallas_call(
        _argmin_tc_body,
        grid=(m // BM,),
        in_specs=[
            pl.BlockSpec((BM, D), lambda i: (i, 0)),
            pl.BlockSpec((K, D), lambda i: (0, 0)),
            pl.BlockSpec((1, K), lambda i: (0, 0)),
        ],
        out_specs=pl.BlockSpec((BM,), lambda i: (i,)),
        out_shape=jax.ShapeDtypeStruct((m,), jnp.int32),
        scratch_shapes=[pltpu.VMEM((1, K), jnp.float32)],
        compiler_params=pltpu.CompilerParams(
            dimension_semantics=("arbitrary",)),
    )(q, codebook, iota)


def _make_gather(b_total):
    info = plsc.get_sparse_core_info()
    nc, ns = info.num_cores, info.num_subcores
    nw = nc * ns
    b_per_w = b_total // nw
    chunk = 128
    n_chunks = b_per_w // chunk
    mesh = plsc.VectorSubcoreMesh(core_axis_name="c", subcore_axis_name="s")

    @functools.partial(
        pl.kernel, mesh=mesh,
        out_type=jax.ShapeDtypeStruct((b_total, D), jnp.float32),
        scratch_types=[
            pltpu.VMEM((b_per_w,), jnp.int32),
            pltpu.VMEM((chunk, D), jnp.float32),
            pltpu.VMEM((chunk, D), jnp.float32),
            pltpu.SemaphoreType.DMA,
            pltpu.SemaphoreType.DMA,
        ],
    )
    def gather_k(table_hbm, idx_hbm, out_hbm, idx_v, rows0, rows1, sem0, sem1):
        wid = lax.axis_index("s") * nc + lax.axis_index("c")
        base = wid * b_per_w
        pltpu.sync_copy(idx_hbm.at[pl.ds(base, b_per_w)], idx_v)
        bufs = (rows0, rows1)
        sems = (sem0, sem1)

        def start(i):
            return pltpu.async_copy(
                table_hbm.at[idx_v.at[pl.ds(i * chunk, chunk)]],
                bufs[i % 2], sems[i % 2])

        cps = {0: start(0)}
        for i in range(n_chunks):
            if i + 1 < n_chunks:
                cps[i + 1] = start(i + 1)
            cps[i].wait()
            pltpu.sync_copy(bufs[i % 2],
                            out_hbm.at[pl.ds(base + i * chunk, chunk)])

    return gather_k


NSPLIT = 1


def kernel(ze, codebook):
    b, t, f = ze.shape
    m = b * t * (f // D)
    q = ze.reshape(m, D)
    ms = m // NSPLIT
    gather = _make_gather(ms)
    parts = []
    for s in range(NSPLIT):
        qs = lax.slice_in_dim(q, s * ms, (s + 1) * ms, axis=0)
        idx = _argmin_indices(qs, codebook)
        parts.append(gather(codebook, idx))
    zq = parts[0] if len(parts) == 1 else jnp.concatenate(parts, axis=0)
    return zq.reshape(b, t, f)


# speedup vs baseline: 3.2708x; 1.0035x over previous
import functools

import jax
import jax.numpy as jnp
from jax import lax
from jax.experimental import pallas as pl
from jax.experimental.pallas import tpu as pltpu
from jax.experimental.pallas import tpu_sc as plsc

K = 8192
D = 256
BM = 1024


def _csq_body(cb_ref, csq_ref):
    cb = cb_ref[...]
    csq_ref[...] = jnp.sum(cb * cb, axis=1)[None, :]


def _argmin_tc_body(q_ref, cb_ref, iota_ref, idx_ref, csq_ref):
    @pl.when(pl.program_id(0) == 0)
    def _():
        cb = cb_ref[...]
        csq_ref[...] = jnp.sum(cb * cb, axis=1)[None, :]

    q = q_ref[...]
    scores = lax.dot_general(
        q, cb_ref[...], (((1,), (1,)), ((), ())),
        preferred_element_type=jnp.float32)
    zsq = jnp.sum(q * q, axis=1, keepdims=True)
    d = (zsq + csq_ref[...]) - 2.0 * scores
    m = jnp.min(d, axis=1, keepdims=True)
    idx_f = jnp.min(jnp.where(d == m, iota_ref[...], jnp.float32(K)), axis=1)
    idx_ref[...] = idx_f.astype(jnp.int32)


def _codebook_norms(codebook):
    return pl.pallas_call(
        _csq_body,
        out_shape=jax.ShapeDtypeStruct((1, K), jnp.float32),
    )(codebook)


def _argmin_indices(q, codebook):
    m = q.shape[0]
    iota = jnp.arange(K, dtype=jnp.float32)[None, :]
    return pl.pallas_call(
        _argmin_tc_body,
        grid=(m // BM,),
        in_specs=[
            pl.BlockSpec((BM, D), lambda i: (i, 0)),
            pl.BlockSpec((K, D), lambda i: (0, 0)),
            pl.BlockSpec((1, K), lambda i: (0, 0)),
        ],
        out_specs=pl.BlockSpec((BM,), lambda i: (i,)),
        out_shape=jax.ShapeDtypeStruct((m,), jnp.int32),
        scratch_shapes=[pltpu.VMEM((1, K), jnp.float32)],
        compiler_params=pltpu.CompilerParams(
            dimension_semantics=("arbitrary",)),
    )(q, codebook, iota)


def _make_gather(b_total):
    info = plsc.get_sparse_core_info()
    nc, ns = info.num_cores, info.num_subcores
    nw = nc * ns
    b_per_w = b_total // nw
    chunk = 128
    n_chunks = b_per_w // chunk
    mesh = plsc.VectorSubcoreMesh(core_axis_name="c", subcore_axis_name="s")

    @functools.partial(
        pl.kernel, mesh=mesh,
        out_type=jax.ShapeDtypeStruct((b_total, D), jnp.float32),
        scratch_types=[
            pltpu.VMEM((b_per_w,), jnp.int32),
            pltpu.VMEM((chunk, D), jnp.float32),
            pltpu.VMEM((chunk, D), jnp.float32),
            pltpu.VMEM((chunk, D), jnp.float32),
            pltpu.SemaphoreType.DMA,
            pltpu.SemaphoreType.DMA,
            pltpu.SemaphoreType.DMA,
            pltpu.SemaphoreType.DMA,
            pltpu.SemaphoreType.DMA,
            pltpu.SemaphoreType.DMA,
        ],
    )
    def gather_k(table_hbm, idx_hbm, out_hbm, idx_v,
                 rows0, rows1, rows2, g0, g1, g2, s0, s1, s2):
        wid = lax.axis_index("s") * nc + lax.axis_index("c")
        base = wid * b_per_w
        pltpu.sync_copy(idx_hbm.at[pl.ds(base, b_per_w)], idx_v)
        bufs = (rows0, rows1, rows2)
        gsems = (g0, g1, g2)
        ssems = (s0, s1, s2)

        def gstart(i):
            return pltpu.async_copy(
                table_hbm.at[idx_v.at[pl.ds(i * chunk, chunk)]],
                bufs[i % 3], gsems[i % 3])

        def sstart(i):
            return pltpu.async_copy(
                bufs[i % 3], out_hbm.at[pl.ds(base + i * chunk, chunk)],
                ssems[i % 3])

        cps = {0: gstart(0)}
        if n_chunks > 1:
            cps[1] = gstart(1)
        scp = {}
        for i in range(n_chunks):
            if i >= 1:
                scp[i - 1].wait()
            if i + 2 < n_chunks:
                cps[i + 2] = gstart(i + 2)
            cps[i].wait()
            scp[i] = sstart(i)
        scp[n_chunks - 1].wait()

    return gather_k


NSPLIT = 1


def kernel(ze, codebook):
    b, t, f = ze.shape
    m = b * t * (f // D)
    q = ze.reshape(m, D)
    ms = m // NSPLIT
    gather = _make_gather(ms)
    parts = []
    for s in range(NSPLIT):
        qs = lax.slice_in_dim(q, s * ms, (s + 1) * ms, axis=0)
        idx = _argmin_indices(qs, codebook)
        parts.append(gather(codebook, idx))
    zq = parts[0] if len(parts) == 1 else jnp.concatenate(parts, axis=0)
    return zq.reshape(b, t, f)


# speedup vs baseline: 3.2865x; 1.0048x over previous
import functools

import jax
import jax.numpy as jnp
from jax import lax
from jax.experimental import pallas as pl
from jax.experimental.pallas import tpu as pltpu
from jax.experimental.pallas import tpu_sc as plsc

K = 8192
D = 256
BM = 1024


def _argmin_tc_body(q_ref, cb_ref, iota_ref, idx_ref, csq_ref):
    @pl.when(pl.program_id(0) == 0)
    def _():
        cb = cb_ref[...]
        csq_ref[...] = jnp.sum(cb * cb, axis=1)[None, :]

    q = q_ref[...]
    scores = lax.dot_general(
        q, cb_ref[...], (((1,), (1,)), ((), ())),
        preferred_element_type=jnp.float32)
    zsq = jnp.sum(q * q, axis=1, keepdims=True)
    d = (zsq + csq_ref[...]) - 2.0 * scores
    m = jnp.min(d, axis=1, keepdims=True)
    idx_f = jnp.min(jnp.where(d == m, iota_ref[...], jnp.float32(K)), axis=1)
    idx_ref[...] = idx_f.astype(jnp.int32)


def _argmin_indices(q, codebook):
    m = q.shape[0]
    iota = jnp.arange(K, dtype=jnp.float32)[None, :]
    return pl.pallas_call(
        _argmin_tc_body,
        grid=(m // BM,),
        in_specs=[
            pl.BlockSpec((BM, D), lambda i: (i, 0)),
            pl.BlockSpec((K, D), lambda i: (0, 0)),
            pl.BlockSpec((1, K), lambda i: (0, 0)),
        ],
        out_specs=pl.BlockSpec((BM,), lambda i: (i,)),
        out_shape=jax.ShapeDtypeStruct((m,), jnp.int32),
        scratch_shapes=[pltpu.VMEM((1, K), jnp.float32)],
        compiler_params=pltpu.CompilerParams(
            dimension_semantics=("arbitrary",)),
    )(q, codebook, iota)


def _make_gather(b_total):
    info = plsc.get_sparse_core_info()
    nc, ns = info.num_cores, info.num_subcores
    nw = nc * ns
    b_per_w = b_total // nw
    chunk = 128
    n_chunks = b_per_w // chunk
    mesh = plsc.VectorSubcoreMesh(core_axis_name="c", subcore_axis_name="s")

    @functools.partial(
        pl.kernel, mesh=mesh,
        out_type=jax.ShapeDtypeStruct((b_total, D), jnp.float32),
        scratch_types=[
            pltpu.VMEM((b_per_w,), jnp.int32),
            pltpu.VMEM((chunk, D), jnp.float32),
            pltpu.VMEM((chunk, D), jnp.float32),
            pltpu.VMEM((chunk, D), jnp.float32),
            pltpu.SemaphoreType.DMA,
            pltpu.SemaphoreType.DMA,
            pltpu.SemaphoreType.DMA,
            pltpu.SemaphoreType.DMA,
            pltpu.SemaphoreType.DMA,
            pltpu.SemaphoreType.DMA,
        ],
    )
    def gather_k(table_hbm, idx_hbm, out_hbm, idx_v,
                 rows0, rows1, rows2, g0, g1, g2, s0, s1, s2):
        wid = lax.axis_index("s") * nc + lax.axis_index("c")
        base = wid * b_per_w
        pltpu.sync_copy(idx_hbm.at[pl.ds(base, b_per_w)], idx_v)
        bufs = (rows0, rows1, rows2)
        gsems = (g0, g1, g2)
        ssems = (s0, s1, s2)

        def gstart(i):
            return pltpu.async_copy(
                table_hbm.at[idx_v.at[pl.ds(i * chunk, chunk)]],
                bufs[i % 3], gsems[i % 3])

        def sstart(i):
            return pltpu.async_copy(
                bufs[i % 3], out_hbm.at[pl.ds(base + i * chunk, chunk)],
                ssems[i % 3])

        cps = {0: gstart(0)}
        if n_chunks > 1:
            cps[1] = gstart(1)
        scp = {}
        for i in range(n_chunks):
            if i >= 1:
                scp[i - 1].wait()
            if i + 2 < n_chunks:
                cps[i + 2] = gstart(i + 2)
            cps[i].wait()
            scp[i] = sstart(i)
        scp[n_chunks - 1].wait()

    return gather_k


NSPLIT = 1


def kernel(ze, codebook):
    b, t, f = ze.shape
    m = b * t * (f // D)
    q = ze.reshape(m, D)
    ms = m // NSPLIT
    gather = _make_gather(ms)
    parts = []
    for s in range(NSPLIT):
        qs = lax.slice_in_dim(q, s * ms, (s + 1) * ms, axis=0)
        idx = _argmin_indices(qs, codebook)
        parts.append(gather(codebook, idx))
    zq = parts[0] if len(parts) == 1 else jnp.concatenate(parts, axis=0)
    return zq.reshape(b, t, f)
